# baseline (device time: 25790 ns/iter reference)
import jax
import jax.numpy as jnp
from jax import lax
from jax.experimental import pallas as pl
from jax.experimental.pallas import tpu as pltpu

N_Z = 4
B, H, D, BS = 8, 8, 64, 16
PAGES_PER_SHARD = 64
KEYS = PAGES_PER_SHARD * BS
NSLOT = 64
SCALE = D ** -0.5
NEG = -1e30


def kernel(Q, K, V, bt, lens):

    def body(q_ref, k_ref, v_ref, bt_ref, lens_ref, out_ref,
             comm_ref, kt_ref, vt_ref, w_ref,
             kv_sems, send_sems, recv_sems):
        my_x = lax.axis_index("x")
        my_y = lax.axis_index("y")
        my_z = lax.axis_index("z")

        kv_copies = []
        for h in range(H):
            ck = pltpu.make_async_copy(
                k_ref.at[:, :, h, :], kt_ref.at[h], kv_sems.at[0, h])
            cv = pltpu.make_async_copy(
                v_ref.at[:, :, h, :], vt_ref.at[h], kv_sems.at[1, h])
            ck.start()
            cv.start()
            kv_copies.append((ck, cv))

        z_off = my_z * PAGES_PER_SHARD
        pid_f = (z_off
                 + lax.broadcasted_iota(jnp.int32, (NSLOT, KEYS), 1) // BS
                 ).astype(jnp.float32)
        slot_iota = lax.broadcasted_iota(jnp.int32, (NSLOT, KEYS), 0)

        ii = lax.broadcasted_iota(jnp.int32, (NSLOT, NSLOT), 0)
        jj = lax.broadcasted_iota(jnp.int32, (NSLOT, NSLOT), 1)
        ident = jnp.where(ii == jj, 1.0, 0.0)
        btT = lax.dot_general(
            ident, bt_ref[:, :].astype(jnp.float32), (((1,), (1,)), ((), ())),
            preferred_element_type=jnp.float32)

        for b in range(B):
            bt_col = btT[:, b:b + 1]
            used = jnp.logical_and(bt_col == pid_f, slot_iota < lens_ref[b])
            w_ref[b:b + 1, :] = jnp.sum(
                jnp.where(used, 1.0, 0.0), axis=0, keepdims=True)

        w = w_ref[:, :]
        has = w > 0.0
        for h in range(H):
            ck, cv = kv_copies[h]
            ck.wait()
            cv.wait()
            q_h = q_ref[:, 0, h, :]
            k_h = kt_ref[h].reshape(KEYS, D)
            v_h = vt_ref[h].reshape(KEYS, D)
            s = lax.dot_general(
                q_h, k_h, (((1,), (1,)), ((), ())),
                preferred_element_type=jnp.float32) * SCALE
            s = jnp.where(has, s, NEG)
            m = jnp.max(s, axis=1, keepdims=True)
            e = jnp.exp(s - m) * w
            l = jnp.sum(e, axis=1, keepdims=True)
            o = lax.dot_general(
                e, v_h, (((1,), (0,)), ((), ())),
                preferred_element_type=jnp.float32)
            comm_ref[0, 0, :, h, :] = o
            comm_ref[0, 1, :, h, :] = jnp.broadcast_to(m, (B, D))
            comm_ref[0, 2, :, h, :] = jnp.broadcast_to(l, (B, D))

        barrier_sem = pltpu.get_barrier_semaphore()
        for d in (1, 2, 3):
            pl.semaphore_signal(
                barrier_sem, inc=1,
                device_id=(my_x, my_y, (my_z + d) % N_Z),
                device_id_type=pl.DeviceIdType.MESH,
            )
        pl.semaphore_wait(barrier_sem, 3)

        rdmas = []
        for d in (1, 2, 3):
            dst_slot = N_Z - d
            rdma = pltpu.make_async_remote_copy(
                src_ref=comm_ref.at[0],
                dst_ref=comm_ref.at[dst_slot],
                send_sem=send_sems.at[d - 1],
                recv_sem=recv_sems.at[dst_slot - 1],
                device_id=(my_x, my_y, (my_z + d) % N_Z),
                device_id_type=pl.DeviceIdType.MESH,
            )
            rdma.start()
            rdmas.append(rdma)
        for rdma in rdmas:
            rdma.wait_recv()
        for rdma in rdmas:
            rdma.wait_send()

        o_all = comm_ref[:, 0]
        m_all = comm_ref[:, 1]
        l_all = comm_ref[:, 2]
        m_max = jnp.max(m_all, axis=0)
        alpha = jnp.exp(m_all - m_max[None])
        l_tot = jnp.sum(l_all * alpha, axis=0)
        out_ref[:, 0, :, :] = jnp.sum(o_all * alpha, axis=0) / l_tot

    return pl.pallas_call(
        body,
        out_shape=jax.ShapeDtypeStruct((B, 1, H, D), jnp.float32),
        in_specs=[
            pl.BlockSpec(memory_space=pltpu.VMEM),
            pl.BlockSpec(memory_space=pltpu.VMEM),
            pl.BlockSpec(memory_space=pltpu.VMEM),
            pl.BlockSpec(memory_space=pltpu.VMEM),
            pl.BlockSpec(memory_space=pltpu.SMEM),
        ],
        out_specs=pl.BlockSpec(memory_space=pltpu.VMEM),
        scratch_shapes=[
            pltpu.VMEM((N_Z, 3, B, H, D), jnp.float32),
            pltpu.VMEM((H, PAGES_PER_SHARD, BS, D), jnp.float32),
            pltpu.VMEM((H, PAGES_PER_SHARD, BS, D), jnp.float32),
            pltpu.VMEM((B, KEYS), jnp.float32),
            pltpu.SemaphoreType.DMA((2, H)),
            pltpu.SemaphoreType.DMA((3,)),
            pltpu.SemaphoreType.DMA((3,)),
        ],
        compiler_params=pltpu.CompilerParams(collective_id=0),
    )(Q, K, V, bt, lens)


# device time: 18298 ns/iter; 1.4094x vs baseline; 1.4094x over previous
import jax
import jax.numpy as jnp
from jax import lax
from jax.experimental import pallas as pl
from jax.experimental.pallas import tpu as pltpu

N_Z = 4
B, H, D, BS = 8, 8, 64, 16
PAGES_PER_SHARD = 64
KEYS = PAGES_PER_SHARD * BS
NSLOT = 64
SCALE = D ** -0.5
NEG = -1e30


def kernel(Q, K, V, bt, lens):
    K_t = jnp.transpose(K.reshape(KEYS, H, D), (1, 0, 2)).astype(jnp.bfloat16)
    V_t = jnp.transpose(V.reshape(KEYS, H, D), (1, 0, 2)).astype(jnp.bfloat16)

    def body(q_ref, k_ref, v_ref, bt_ref, lens_ref, out_ref,
             comm_ref, w_ref, send_sems, recv_sems):
        my_x = lax.axis_index("x")
        my_y = lax.axis_index("y")
        my_z = lax.axis_index("z")

        z_off = my_z * PAGES_PER_SHARD
        pid_f = (z_off
                 + lax.broadcasted_iota(jnp.int32, (NSLOT, KEYS), 1) // BS
                 ).astype(jnp.float32)
        slot_iota = lax.broadcasted_iota(jnp.int32, (NSLOT, KEYS), 0)

        ii = lax.broadcasted_iota(jnp.int32, (NSLOT, NSLOT), 0)
        jj = lax.broadcasted_iota(jnp.int32, (NSLOT, NSLOT), 1)
        ident = jnp.where(ii == jj, 1.0, 0.0)
        btT = lax.dot_general(
            ident, bt_ref[:, :].astype(jnp.float32), (((1,), (1,)), ((), ())),
            preferred_element_type=jnp.float32)

        for b in range(B):
            bt_col = btT[:, b:b + 1]
            used = jnp.logical_and(bt_col == pid_f, slot_iota < lens_ref[b])
            w_ref[b:b + 1, :] = jnp.sum(
                jnp.where(used, 1.0, 0.0), axis=0, keepdims=True)

        w = w_ref[:, :]
        has = w > 0.0
        for h in range(H):
            q_h = q_ref[:, 0, h, :].astype(jnp.bfloat16)
            s = lax.dot_general(
                q_h, k_ref[h], (((1,), (1,)), ((), ())),
                preferred_element_type=jnp.float32) * SCALE
            s = jnp.where(has, s, NEG)
            m = jnp.max(s, axis=1, keepdims=True)
            e = jnp.exp(s - m) * w
            l = jnp.sum(e, axis=1, keepdims=True)
            o = lax.dot_general(
                e.astype(jnp.bfloat16), v_ref[h], (((1,), (0,)), ((), ())),
                preferred_element_type=jnp.float32)
            comm_ref[0, 0, :, h, :] = o
            comm_ref[0, 1, :, h, :] = jnp.broadcast_to(m, (B, D))
            comm_ref[0, 2, :, h, :] = jnp.broadcast_to(l, (B, D))

        barrier_sem = pltpu.get_barrier_semaphore()
        for d in (1, 2, 3):
            pl.semaphore_signal(
                barrier_sem, inc=1,
                device_id=(my_x, my_y, (my_z + d) % N_Z),
                device_id_type=pl.DeviceIdType.MESH,
            )
        pl.semaphore_wait(barrier_sem, 3)

        rdmas = []
        for d in (1, 2, 3):
            dst_slot = N_Z - d
            rdma = pltpu.make_async_remote_copy(
                src_ref=comm_ref.at[0],
                dst_ref=comm_ref.at[dst_slot],
                send_sem=send_sems.at[d - 1],
                recv_sem=recv_sems.at[dst_slot - 1],
                device_id=(my_x, my_y, (my_z + d) % N_Z),
                device_id_type=pl.DeviceIdType.MESH,
            )
            rdma.start()
            rdmas.append(rdma)
        for rdma in rdmas:
            rdma.wait_recv()
        for rdma in rdmas:
            rdma.wait_send()

        o_all = comm_ref[:, 0]
        m_all = comm_ref[:, 1]
        l_all = comm_ref[:, 2]
        m_max = jnp.max(m_all, axis=0)
        alpha = jnp.exp(m_all - m_max[None])
        l_tot = jnp.sum(l_all * alpha, axis=0)
        out_ref[:, 0, :, :] = jnp.sum(o_all * alpha, axis=0) / l_tot

    return pl.pallas_call(
        body,
        out_shape=jax.ShapeDtypeStruct((B, 1, H, D), jnp.float32),
        in_specs=[
            pl.BlockSpec(memory_space=pltpu.VMEM),
            pl.BlockSpec(memory_space=pltpu.VMEM),
            pl.BlockSpec(memory_space=pltpu.VMEM),
            pl.BlockSpec(memory_space=pltpu.VMEM),
            pl.BlockSpec(memory_space=pltpu.SMEM),
        ],
        out_specs=pl.BlockSpec(memory_space=pltpu.VMEM),
        scratch_shapes=[
            pltpu.VMEM((N_Z, 3, B, H, D), jnp.float32),
            pltpu.VMEM((B, KEYS), jnp.float32),
            pltpu.SemaphoreType.DMA((3,)),
            pltpu.SemaphoreType.DMA((3,)),
        ],
        compiler_params=pltpu.CompilerParams(collective_id=0),
    )(Q, K_t, V_t, bt, lens)


# device time: 14043 ns/iter; 1.8365x vs baseline; 1.3030x over previous
import jax
import jax.numpy as jnp
from jax import lax
from jax.experimental import pallas as pl
from jax.experimental.pallas import tpu as pltpu

N_Z = 4
B, H, D, BS = 8, 8, 64, 16
PAGES_PER_SHARD = 64
KEYS = PAGES_PER_SHARD * BS
NSLOT = 64
SCALE = D ** -0.5
NEG = -1e30


def kernel(Q, K, V, bt, lens):
    K_t = jnp.transpose(K.reshape(KEYS, H, D), (1, 0, 2))
    V_t = jnp.transpose(V.reshape(KEYS, H, D), (1, 0, 2))

    def body(q_ref, k_ref, v_ref, bt_ref, lens_ref, out_ref,
             o_ref, s_ref, w_ref, send_sems, recv_sems):
        my_x = lax.axis_index("x")
        my_y = lax.axis_index("y")
        my_z = lax.axis_index("z")

        barrier_sem = pltpu.get_barrier_semaphore()
        for d in (1, 2, 3):
            pl.semaphore_signal(
                barrier_sem, inc=1,
                device_id=(my_x, my_y, (my_z + d) % N_Z),
                device_id_type=pl.DeviceIdType.MESH,
            )

        z_off = my_z * PAGES_PER_SHARD
        pid_f = (z_off
                 + lax.broadcasted_iota(jnp.int32, (NSLOT, KEYS), 1) // BS
                 ).astype(jnp.float32)
        slot_iota = lax.broadcasted_iota(jnp.int32, (NSLOT, KEYS), 0)

        ii = lax.broadcasted_iota(jnp.int32, (NSLOT, NSLOT), 0)
        jj = lax.broadcasted_iota(jnp.int32, (NSLOT, NSLOT), 1)
        ident = jnp.where(ii == jj, 1.0, 0.0)
        btT = lax.dot_general(
            ident, bt_ref[:, :].astype(jnp.float32), (((1,), (1,)), ((), ())),
            preferred_element_type=jnp.float32)

        for b in range(B):
            bt_col = btT[:, b:b + 1]
            used = jnp.logical_and(bt_col == pid_f, slot_iota < lens_ref[b])
            w_ref[b:b + 1, :] = jnp.sum(
                jnp.where(used, 1.0, 0.0), axis=0, keepdims=True)

        w = w_ref[:, :]
        has = w > 0.0
        for h in range(H):
            q_h = q_ref[:, 0, h, :]
            s = lax.dot_general(
                q_h, k_ref[h], (((1,), (1,)), ((), ())),
                preferred_element_type=jnp.float32) * SCALE
            s = jnp.where(has, s, NEG)
            m = jnp.max(s, axis=1, keepdims=True)
            e = jnp.exp(s - m) * w
            l = jnp.sum(e, axis=1, keepdims=True)
            o = lax.dot_general(
                e, v_ref[h], (((1,), (0,)), ((), ())),
                preferred_element_type=jnp.float32)
            o_ref[0, :, h, :] = o
            s_ref[0, 0, :, h:h + 1] = m
            s_ref[0, 1, :, h:h + 1] = l

        pl.semaphore_wait(barrier_sem, 3)

        rdmas = []
        for d in (1, 2, 3):
            dst_slot = N_Z - d
            peer = (my_x, my_y, (my_z + d) % N_Z)
            for i, buf in enumerate((o_ref, s_ref)):
                rdma = pltpu.make_async_remote_copy(
                    src_ref=buf.at[0],
                    dst_ref=buf.at[dst_slot],
                    send_sem=send_sems.at[i, d - 1],
                    recv_sem=recv_sems.at[i, dst_slot - 1],
                    device_id=peer,
                    device_id_type=pl.DeviceIdType.MESH,
                )
                rdma.start()
                rdmas.append(rdma)
        for rdma in rdmas:
            rdma.wait_recv()
        for rdma in rdmas:
            rdma.wait_send()

        for h in range(H):
            m_all = s_ref[:, 0, :, h:h + 1]
            l_all = s_ref[:, 1, :, h:h + 1]
            m_max = jnp.max(m_all, axis=0)
            alpha = jnp.exp(m_all - m_max[None])
            l_tot = jnp.sum(l_all * alpha, axis=0)
            o_h = jnp.sum(o_ref[:, :, h, :] * alpha, axis=0)
            out_ref[:, 0, h, :] = o_h / l_tot

    return pl.pallas_call(
        body,
        out_shape=jax.ShapeDtypeStruct((B, 1, H, D), jnp.float32),
        in_specs=[
            pl.BlockSpec(memory_space=pltpu.VMEM),
            pl.BlockSpec(memory_space=pltpu.VMEM),
            pl.BlockSpec(memory_space=pltpu.VMEM),
            pl.BlockSpec(memory_space=pltpu.VMEM),
            pl.BlockSpec(memory_space=pltpu.SMEM),
        ],
        out_specs=pl.BlockSpec(memory_space=pltpu.VMEM),
        scratch_shapes=[
            pltpu.VMEM((N_Z, B, H, D), jnp.float32),
            pltpu.VMEM((N_Z, 2, B, H), jnp.float32),
            pltpu.VMEM((B, KEYS), jnp.float32),
            pltpu.SemaphoreType.DMA((2, 3)),
            pltpu.SemaphoreType.DMA((2, 3)),
        ],
        compiler_params=pltpu.CompilerParams(collective_id=0),
    )(Q, K_t, V_t, bt, lens)
